# baseline (speedup 1.0000x reference)
import jax
import jax.numpy as jnp
from jax import lax
from jax.experimental import pallas as pl
from jax.experimental.pallas import tpu as pltpu
from jax.experimental.pallas import tpu_sc as plsc

B = 1024
L = 50
D = 128
E = 8
NW = 32
RPW = B // NW
NB = 4

_info = plsc.get_sparse_core_info()
_NC, _NS = _info.num_cores, _info.num_subcores


def _body(idx_hbm, loc_hbm, fixvec_hbm, table_hbm, out_hbm,
          idx_v, loc_v, fix_v, b0, b1, b2, b3,
          g0, g1, g2, g3, w0, w1, w2, w3):
    bufs = (b0, b1, b2, b3)
    gs = (g0, g1, g2, g3)
    ws = (w0, w1, w2, w3)
    wid = lax.axis_index("s") * _NC + lax.axis_index("c")
    obase = wid * RPW
    pltpu.sync_copy(idx_hbm.at[wid], idx_v)
    pltpu.sync_copy(loc_hbm.at[wid], loc_v)
    pltpu.sync_copy(fixvec_hbm, fix_v)
    cols = [lax.broadcasted_iota(jnp.int32, (16,), 0) + jnp.int32(16 * c)
            for c in range(D // 16)]

    def gather(r, slot):
        return pltpu.async_copy(table_hbm.at[idx_v.at[r]], bufs[slot],
                                gs[slot])

    def patch(r, slot):
        lvec = loc_v[r, :]
        for j in range(E):
            rows = lax.gather(
                lvec, jnp.full((16, 1), j, dtype=jnp.int32),
                lax.GatherDimensionNumbers(
                    offset_dims=(), collapsed_slice_dims=(0,),
                    start_index_map=(0,)),
                (1,), mode=lax.GatherScatterMode.PROMISE_IN_BOUNDS)
            for c in range(D // 16):
                val = fix_v[jnp.int32(j), pl.ds(16 * c, 16)]
                plsc.store_scatter(bufs[slot], [rows, cols[c]], val)

    def step(r, slot, wait_prev_write):
        pltpu.make_async_copy(table_hbm.at[idx_v.at[r]], bufs[slot],
                              gs[slot]).wait()
        patch(r, slot)
        pltpu.async_copy(bufs[slot], out_hbm.at[:, obase + r], ws[slot])
        nslot = (slot + NB - 1) % NB
        if wait_prev_write:
            pltpu.make_async_copy(bufs[nslot], out_hbm.at[:, obase],
                                  ws[nslot]).wait()
        gather(r + NB - 1, nslot)

    for r in range(NB - 1):
        gather(jnp.int32(r), r)
    step(jnp.int32(0), 0, False)

    def outer(i, carry):
        ii = i.astype(jnp.int32)
        for b in range(NB):
            step(jnp.int32(1 + b) + ii * jnp.int32(NB), (1 + b) % NB, True)
        return carry
    lax.fori_loop(jnp.int32(0), jnp.int32((RPW - NB) // NB), outer,
                  jnp.int32(0))

    for r in range(RPW - NB + 1, RPW):
        slot = r % NB
        pltpu.make_async_copy(table_hbm.at[idx_v.at[jnp.int32(r)]],
                              bufs[slot], gs[slot]).wait()
        patch(jnp.int32(r), slot)
        pltpu.async_copy(bufs[slot], out_hbm.at[:, obase + jnp.int32(r)],
                         ws[slot])
    for r in range(RPW - NB, RPW):
        slot = r % NB
        pltpu.make_async_copy(bufs[slot], out_hbm.at[:, obase],
                              ws[slot]).wait()


def kernel(input_ids, fix_offsets, table, fix_vec):
    start = fix_offsets.astype(jnp.int32) + 1
    rel = jnp.arange(L, dtype=jnp.int32)[None, :] - start[:, None]
    ids32 = jnp.where((rel >= 0) & (rel < E), 0, input_ids.astype(jnp.int32))
    idx = ids32.reshape(NW, RPW, L)
    loc = (start[:, None] + jnp.arange(16, dtype=jnp.int32)[None, :]
           ).reshape(NW, RPW, 16)
    mesh = plsc.VectorSubcoreMesh(core_axis_name="c", subcore_axis_name="s")
    run = pl.kernel(
        _body,
        mesh=mesh,
        out_type=jax.ShapeDtypeStruct((L, B, D), jnp.float32),
        scratch_types=(
            [pltpu.VMEM((RPW, L), jnp.int32),
             pltpu.VMEM((RPW, 16), jnp.int32),
             pltpu.VMEM((E, D), jnp.float32)]
            + [pltpu.VMEM((L, D), jnp.float32)] * NB
            + [pltpu.SemaphoreType.DMA] * (2 * NB)
        ),
        compiler_params=pltpu.CompilerParams(
            needs_layout_passes=False, use_tc_tiling_on_sc=False),
    )
    out = run(idx, loc, fix_vec, table)
    return out.transpose(1, 0, 2)


# speedup vs baseline: 8.6643x; 8.6643x over previous
import jax
import jax.numpy as jnp
from jax import lax
from jax.experimental import pallas as pl
from jax.experimental.pallas import tpu as pltpu
from jax.experimental.pallas import tpu_sc as plsc

B = 1024
L = 50
D = 128
E = 8
NW = 32
RPW = B // NW
NB = 8

_info = plsc.get_sparse_core_info()
_NC, _NS = _info.num_cores, _info.num_subcores


def _body(idx_hbm, loc_hbm, fixvec_hbm, table_hbm, out_hbm,
          idx_v, loc_v, fix_v, b0, b1, b2, b3, b4, b5, b6, b7,
          g0, g1, g2, g3, g4, g5, g6, g7, w0, w1, w2, w3, w4, w5, w6, w7):
    bufs = (b0, b1, b2, b3, b4, b5, b6, b7)
    gs = (g0, g1, g2, g3, g4, g5, g6, g7)
    ws = (w0, w1, w2, w3, w4, w5, w6, w7)
    wid = lax.axis_index("s") * _NC + lax.axis_index("c")
    obase = wid * RPW
    pltpu.sync_copy(idx_hbm.at[wid], idx_v)
    pltpu.sync_copy(loc_hbm.at[wid], loc_v)
    pltpu.sync_copy(fixvec_hbm, fix_v)
    cols = [lax.broadcasted_iota(jnp.int32, (16,), 0) + jnp.int32(16 * c)
            for c in range(D // 16)]

    def gather(r, slot):
        return pltpu.async_copy(table_hbm.at[idx_v.at[r]], bufs[slot],
                                gs[slot])

    def patch(r, slot):
        lvec = loc_v[r, :]
        for j in range(E):
            rows = lax.gather(
                lvec, jnp.full((16, 1), j, dtype=jnp.int32),
                lax.GatherDimensionNumbers(
                    offset_dims=(), collapsed_slice_dims=(0,),
                    start_index_map=(0,)),
                (1,), mode=lax.GatherScatterMode.PROMISE_IN_BOUNDS)
            for c in range(D // 16):
                val = fix_v[jnp.int32(j), pl.ds(16 * c, 16)]
                plsc.store_scatter(bufs[slot], [rows, cols[c]], val)

    def step(r, slot, wait_prev_write):
        pltpu.make_async_copy(table_hbm.at[idx_v.at[r]], bufs[slot],
                              gs[slot]).wait()
        patch(r, slot)
        pltpu.async_copy(bufs[slot], out_hbm.at[:, obase + r], ws[slot])
        nslot = (slot + NB - 1) % NB
        if wait_prev_write:
            pltpu.make_async_copy(bufs[nslot], out_hbm.at[:, obase],
                                  ws[nslot]).wait()
        gather(r + NB - 1, nslot)

    for r in range(NB - 1):
        gather(jnp.int32(r), r)
    step(jnp.int32(0), 0, False)

    def outer(i, carry):
        ii = i.astype(jnp.int32)
        for b in range(NB):
            step(jnp.int32(1 + b) + ii * jnp.int32(NB), (1 + b) % NB, True)
        return carry
    lax.fori_loop(jnp.int32(0), jnp.int32((RPW - NB) // NB), outer,
                  jnp.int32(0))

    for r in range(RPW - NB + 1, RPW):
        slot = r % NB
        pltpu.make_async_copy(table_hbm.at[idx_v.at[jnp.int32(r)]],
                              bufs[slot], gs[slot]).wait()
        patch(jnp.int32(r), slot)
        pltpu.async_copy(bufs[slot], out_hbm.at[:, obase + jnp.int32(r)],
                         ws[slot])
    for r in range(RPW - NB, RPW):
        slot = r % NB
        pltpu.make_async_copy(bufs[slot], out_hbm.at[:, obase],
                              ws[slot]).wait()


def kernel(input_ids, fix_offsets, table, fix_vec):
    idx = input_ids.astype(jnp.int32).reshape(NW, RPW, L)
    start = fix_offsets.astype(jnp.int32) + 1
    loc = (start[:, None] + jnp.arange(16, dtype=jnp.int32)[None, :]
           ).reshape(NW, RPW, 16)
    mesh = plsc.VectorSubcoreMesh(core_axis_name="c", subcore_axis_name="s")
    run = pl.kernel(
        _body,
        mesh=mesh,
        out_type=jax.ShapeDtypeStruct((L, B, D), jnp.float32),
        scratch_types=(
            [pltpu.VMEM((RPW, L), jnp.int32),
             pltpu.VMEM((RPW, 16), jnp.int32),
             pltpu.VMEM((E, D), jnp.float32)]
            + [pltpu.VMEM((L, D), jnp.float32)] * NB
            + [pltpu.SemaphoreType.DMA] * (2 * NB)
        ),
        compiler_params=pltpu.CompilerParams(
            needs_layout_passes=False, use_tc_tiling_on_sc=False),
    )
    out = run(idx, loc, fix_vec, table)
    return out.transpose(1, 0, 2)


# speedup vs baseline: 8.6727x; 1.0010x over previous
import jax
import jax.numpy as jnp
from jax import lax
from jax.experimental import pallas as pl
from jax.experimental.pallas import tpu as pltpu
from jax.experimental.pallas import tpu_sc as plsc

B = 1024
L = 50
D = 128
E = 8
NW = 32
RPW = B // NW
NB = 4

_info = plsc.get_sparse_core_info()
_NC, _NS = _info.num_cores, _info.num_subcores


def _body(idx_hbm, offs_hbm, fixvec_hbm, table_hbm, out_hbm,
          idx_v, offs_v, fix_v, b0, b1, b2, b3,
          g0, g1, g2, g3, w0, w1, w2, w3):
    bufs = (b0, b1, b2, b3)
    gs = (g0, g1, g2, g3)
    ws = (w0, w1, w2, w3)
    wid = lax.axis_index("s") * _NC + lax.axis_index("c")
    obase = wid * RPW
    pltpu.sync_copy(idx_hbm.at[wid], idx_v)
    pltpu.sync_copy(offs_hbm.at[pl.ds(obase, RPW)], offs_v)
    pltpu.sync_copy(fixvec_hbm, fix_v)
    off_lo = offs_v[pl.ds(0, 16)]
    off_hi = offs_v[pl.ds(16, 16)]
    cols = [lax.broadcasted_iota(jnp.int32, (16,), 0) + jnp.int32(16 * c)
            for c in range(D // 16)]

    def gather(r, slot):
        return pltpu.async_copy(table_hbm.at[idx_v.at[r]], bufs[slot],
                                gs[slot])

    def patch(r, slot):
        half = jnp.where(r < jnp.int32(16), off_lo, off_hi)
        lane = jnp.broadcast_to(lax.rem(r, jnp.int32(16)), (16,)
                                ).reshape(16, 1)
        base = lax.gather(
            half, lane,
            lax.GatherDimensionNumbers(
                offset_dims=(), collapsed_slice_dims=(0,),
                start_index_map=(0,)),
            (1,), mode=lax.GatherScatterMode.PROMISE_IN_BOUNDS)
        for j in range(E):
            rows = base + jnp.int32(j)
            for c in range(D // 16):
                val = fix_v[jnp.int32(j), pl.ds(16 * c, 16)]
                plsc.store_scatter(bufs[slot], [rows, cols[c]], val)

    def step(r, slot, wait_prev_write):
        pltpu.make_async_copy(table_hbm.at[idx_v.at[r]], bufs[slot],
                              gs[slot]).wait()
        patch(r, slot)
        pltpu.async_copy(bufs[slot], out_hbm.at[:, obase + r], ws[slot])
        nslot = (slot + NB - 1) % NB
        if wait_prev_write:
            pltpu.make_async_copy(bufs[nslot], out_hbm.at[:, obase],
                                  ws[nslot]).wait()
        gather(r + NB - 1, nslot)

    for r in range(NB - 1):
        gather(jnp.int32(r), r)
    step(jnp.int32(0), 0, False)

    def outer(i, carry):
        ii = i.astype(jnp.int32)
        for b in range(NB):
            step(jnp.int32(1 + b) + ii * jnp.int32(NB), (1 + b) % NB, True)
        return carry
    lax.fori_loop(jnp.int32(0), jnp.int32((RPW - NB) // NB), outer,
                  jnp.int32(0))

    for r in range(RPW - NB + 1, RPW):
        slot = r % NB
        pltpu.make_async_copy(table_hbm.at[idx_v.at[jnp.int32(r)]],
                              bufs[slot], gs[slot]).wait()
        patch(jnp.int32(r), slot)
        pltpu.async_copy(bufs[slot], out_hbm.at[:, obase + jnp.int32(r)],
                         ws[slot])
    for r in range(RPW - NB, RPW):
        slot = r % NB
        pltpu.make_async_copy(bufs[slot], out_hbm.at[:, obase],
                              ws[slot]).wait()


def kernel(input_ids, fix_offsets, table, fix_vec):
    idx = input_ids.astype(jnp.int32).reshape(NW, RPW, L)
    start = fix_offsets.astype(jnp.int32) + 1
    mesh = plsc.VectorSubcoreMesh(core_axis_name="c", subcore_axis_name="s")
    run = pl.kernel(
        _body,
        mesh=mesh,
        out_type=jax.ShapeDtypeStruct((L, B, D), jnp.float32),
        scratch_types=(
            [pltpu.VMEM((RPW, L), jnp.int32),
             pltpu.VMEM((RPW,), jnp.int32),
             pltpu.VMEM((E, D), jnp.float32)]
            + [pltpu.VMEM((L, D), jnp.float32)] * NB
            + [pltpu.SemaphoreType.DMA] * (2 * NB)
        ),
        compiler_params=pltpu.CompilerParams(
            needs_layout_passes=False, use_tc_tiling_on_sc=False),
    )
    out = run(idx, start, fix_vec, table)
    return out.transpose(1, 0, 2)
